# Initial kernel scaffold; baseline (speedup 1.0000x reference)
import jax
import jax.numpy as jnp
from jax.experimental import pallas as pl


def kernel(sequence_features, params, shared_pentachora, shared_positions):
    raise NotImplementedError("write your pallas kernel here")


# speedup vs baseline: 29.7619x; 29.7619x over previous
import math

import jax
import jax.numpy as jnp
from jax.experimental import pallas as pl

SEQ = 2048
IN_DIM = 1024
P = 256
H = 8
HD = 32
NPENT = 1024
KW = 128
QT = 256
NT = SEQ // QT
SCALE = 1.0 / math.sqrt(HD)
ONE_BITS = 0x40000000


def _gelu(x):
    return 0.5 * x * (1.0 + jax.lax.erf(x * (1.0 / math.sqrt(2.0))))


def _ln(x, g, b):
    mu = jnp.mean(x, axis=-1, keepdims=True)
    xc = x - mu
    var = jnp.mean(xc * xc, axis=-1, keepdims=True)
    return xc / jnp.sqrt(var + 1e-5) * g + b


def _megakernel(x_ref, pentT_ref, pos_ref,
                w_in_ref, b_in_ref, g_in_ref, bt_in_ref,
                w_q_ref, b_q_ref, w_k_ref, b_k_ref, w_v_ref, b_v_ref,
                w_out_ref, b_out_ref,
                wa0_ref, ba0_ref, g0_ref, bb0_ref, wb0_ref, bo0_ref,
                wa1_ref, ba1_ref, g1_ref, bb1_ref, wb1_ref, bo1_ref,
                wa2_ref, ba2_ref, g2_ref, bb2_ref, wb2_ref, bo2_ref,
                out0_ref, out1_ref, out2_ref):
    centT = (pentT_ref[0] + pentT_ref[1] + pentT_ref[2]
             + pentT_ref[3] + pentT_ref[4]) / 5.0
    cn = jnp.sqrt(jnp.sum(centT * centT, axis=0, keepdims=True))
    centT = centT / jnp.maximum(cn, 1e-12)

    pr = jnp.dot(x_ref[...], w_in_ref[...],
                 preferred_element_type=jnp.float32) + b_in_ref[...]
    proj = _gelu(_ln(pr, g_in_ref[...], bt_in_ref[...]))

    n = jnp.sqrt(jnp.sum(proj * proj, axis=-1, keepdims=True))
    fn = proj / jnp.maximum(n, 1e-12)
    sims = jnp.dot(fn, centT, preferred_element_type=jnp.float32)
    rowmax = jnp.max(sims, axis=-1, keepdims=True)
    lane_a = jax.lax.broadcasted_iota(jnp.int32, (SEQ, NPENT), 1)
    aidx = jnp.min(jnp.where(sims == rowmax, lane_a, NPENT),
                   axis=-1, keepdims=True)
    onehot = (lane_a == aidx).astype(jnp.float32)
    pos_col = jnp.sum(onehot * pos_ref[...], axis=-1, keepdims=True)
    pos_row = jnp.transpose(pos_col)

    iota_j = jax.lax.broadcasted_iota(jnp.int32, (SEQ, SEQ), 0)
    iota_c = jax.lax.broadcasted_iota(jnp.int32, (SEQ, SEQ), 1)
    eqm = (pos_col == pos_row) & (iota_j < iota_c)
    rcount_row = jnp.sum(eqm.astype(jnp.int32), axis=0, keepdims=True)

    q = jnp.dot(proj, w_q_ref[...],
                preferred_element_type=jnp.float32) + b_q_ref[...]
    k = jnp.dot(proj, w_k_ref[...],
                preferred_element_type=jnp.float32) + b_k_ref[...]
    v = jnp.dot(proj, w_v_ref[...],
                preferred_element_type=jnp.float32) + b_v_ref[...]

    def tile_body(r0, acc):
        pos_q = pos_col[r0:r0 + QT, :]
        dist = jnp.abs(pos_q - pos_row)
        dbits = jax.lax.bitcast_convert_type(dist, jnp.int32)

        def bs(_, lohi):
            lo, hi = lohi
            mid = lo + (hi - lo) // 2
            cnt = jnp.sum((dbits <= mid).astype(jnp.int32),
                          axis=-1, keepdims=True)
            ge = cnt >= KW
            return (jnp.where(ge, lo, mid + 1), jnp.where(ge, mid, hi))

        lo0 = jnp.zeros((QT, 1), jnp.int32)
        hi0 = jnp.full((QT, 1), ONE_BITS, jnp.int32)
        tbits, _ = jax.lax.fori_loop(0, 31, bs, (lo0, hi0))
        cnt_lt = jnp.sum((dbits < tbits).astype(jnp.int32),
                         axis=-1, keepdims=True)
        rneed = KW - cnt_lt
        sel = (dbits < tbits) | ((dbits == tbits) & (rcount_row < rneed))

        outs = []
        for h in range(H):
            qh = q[r0:r0 + QT, h * HD:(h + 1) * HD]
            kh = k[:, h * HD:(h + 1) * HD]
            vh = v[:, h * HD:(h + 1) * HD]
            s = jax.lax.dot_general(qh, kh, (((1,), (1,)), ((), ())),
                                    preferred_element_type=jnp.float32) * SCALE
            s = jnp.where(sel, s, -1e30)
            m = jnp.max(s, axis=-1, keepdims=True)
            e = jnp.where(sel, jnp.exp(s - m), 0.0)
            p = e / jnp.sum(e, axis=-1, keepdims=True)
            oh = jnp.dot(p, vh, preferred_element_type=jnp.float32)
            outs.append(jnp.sum(oh, axis=0, keepdims=True))
        return acc + jnp.concatenate(outs, axis=-1)

    acc = jnp.zeros((1, P), jnp.float32)
    for t in range(NT):
        acc = tile_body(t * QT, acc)

    pooled = jnp.dot(acc / SEQ, w_out_ref[...],
                     preferred_element_type=jnp.float32) + b_out_ref[...]

    def opinion(wa, ba, g, b, wb, bo):
        hh = jnp.dot(pooled, wa[...],
                     preferred_element_type=jnp.float32) + ba[...]
        hh = _gelu(_ln(hh, g[...], b[...]))
        return jnp.dot(hh, wb[...], preferred_element_type=jnp.float32) + bo[...]

    out0_ref[...] = opinion(wa0_ref, ba0_ref, g0_ref, bb0_ref, wb0_ref, bo0_ref)
    out1_ref[...] = opinion(wa1_ref, ba1_ref, g1_ref, bb1_ref, wb1_ref, bo1_ref)
    out2_ref[...] = opinion(wa2_ref, ba2_ref, g2_ref, bb2_ref, wb2_ref, bo2_ref)


def kernel(sequence_features, params, shared_pentachora, shared_positions):
    x2d = sequence_features[0]
    pentT = jnp.transpose(shared_pentachora, (1, 2, 0))
    pos = shared_positions.reshape(1, NPENT)

    wqkv = params['W_qkv']
    bqkv = params['b_qkv']
    args = [
        x2d, pentT, pos,
        params['W_in'], params['b_in'].reshape(1, P),
        params['ln_in_g'].reshape(1, P), params['ln_in_b'].reshape(1, P),
        wqkv[:, :P], bqkv[:P].reshape(1, P),
        wqkv[:, P:2 * P], bqkv[P:2 * P].reshape(1, P),
        wqkv[:, 2 * P:], bqkv[2 * P:].reshape(1, P),
        params['W_out'], params['b_out'].reshape(1, P),
    ]
    for s in (64, 128, 256):
        args += [
            params[f'W_a_{s}'], params[f'b_a_{s}'].reshape(1, 2 * s),
            params[f'ln_g_{s}'].reshape(1, 2 * s),
            params[f'ln_b_{s}'].reshape(1, 2 * s),
            params[f'W_b_{s}'], params[f'b_b_{s}'].reshape(1, s),
        ]

    o0, o1, o2 = pl.pallas_call(
        _megakernel,
        out_shape=[
            jax.ShapeDtypeStruct((1, 64), jnp.float32),
            jax.ShapeDtypeStruct((1, 128), jnp.float32),
            jax.ShapeDtypeStruct((1, 256), jnp.float32),
        ],
    )(*args)
    return jnp.concatenate([o0, o1, o2], axis=-1)
